# baseline (speedup 1.0000x reference)
import functools

import jax
import jax.numpy as jnp
from jax import lax
from jax.experimental import pallas as pl
from jax.experimental.pallas import tpu as pltpu
from jax.experimental.pallas import tpu_sc as plsc

N = 2097152
NW = 32
PER_W = N // NW
C = 4096
T = PER_W // C

MAX_CELL = 256.0 - 2.0 ** -15

_mesh = plsc.VectorSubcoreMesh(core_axis_name="c", subcore_axis_name="s")


@functools.partial(
    pl.kernel,
    mesh=_mesh,
    out_type=jax.ShapeDtypeStruct((N,), jnp.float32),
    compiler_params=pltpu.CompilerParams(needs_layout_passes=False),
    scratch_types=[
        pltpu.VMEM((C * 3,), jnp.float32),
        pltpu.VMEM((C * 3,), jnp.float32),
        pltpu.VMEM((C,), jnp.int32),
        pltpu.VMEM((C,), jnp.int32),
        pltpu.VMEM((C,), jnp.float32),
        pltpu.VMEM((C,), jnp.float32),
        pltpu.SemaphoreType.DMA,
        pltpu.SemaphoreType.DMA,
        pltpu.SemaphoreType.DMA,
        pltpu.SemaphoreType.DMA,
    ],
)
def _fused_gather(x_hbm, grid_hbm, out_hbm,
                  xv0, xv1, id0, id1, ov0, ov1, sx0, sx1, sg0, sg1):
    wid = lax.axis_index("s") * 2 + lax.axis_index("c")
    base = wid * PER_W
    iota3 = lax.iota(jnp.int32, 16) * 3
    xbuf = (xv0, xv1)
    ibuf = (id0, id1)
    obuf = (ov0, ov1)
    sx = (sx0, sx1)
    sg = (sg0, sg1)

    def xcopy(t):
        return pltpu.make_async_copy(
            x_hbm.at[pl.ds((base + t * C) * 3, C * 3)], xbuf[t % 2], sx[t % 2])

    def gcopy(t):
        return pltpu.make_async_copy(
            grid_hbm.at[ibuf[t % 2]], obuf[t % 2], sg[t % 2])

    def cell(v):
        return jnp.minimum(v * 128.0 + 128.0, MAX_CELL).astype(jnp.int32)

    def compute(t):
        xv, idxv = xbuf[t % 2], ibuf[t % 2]

        def body(g, rows3):
            i = cell(plsc.load_gather(xv, [rows3]))
            j = cell(plsc.load_gather(xv, [rows3 + 1]))
            k = cell(plsc.load_gather(xv, [rows3 + 2]))
            jpart = (j << 7) + ((j >> 3) << 10)
            kpart = (k & 127) | ((k >> 7) << 10)
            idxv[pl.ds(g * 16, 16)] = (i << 16) | jpart | kpart
            return rows3 + 48

        lax.fori_loop(0, C // 16, body, iota3, unroll=4)

    xcopy(0).start()
    for t in range(T):
        xcopy(t).wait()
        if t + 1 < T:
            xcopy(t + 1).start()
        compute(t)
        if t > 0:
            gcopy(t - 1).wait()
            pltpu.sync_copy(obuf[(t - 1) % 2],
                            out_hbm.at[pl.ds(base + (t - 1) * C, C)])
        gcopy(t).start()
    gcopy(T - 1).wait()
    pltpu.sync_copy(obuf[(T - 1) % 2],
                    out_hbm.at[pl.ds(base + (T - 1) * C, C)])


def kernel(x, grid):
    grid_lin = (
        grid.reshape(256, 32, 8, 2, 128)
        .transpose(0, 1, 3, 2, 4)
        .reshape(-1)
    )
    return _fused_gather(x.reshape(-1), grid_lin)


# speedup vs baseline: 17.8458x; 17.8458x over previous
import functools

import jax
import jax.numpy as jnp
from jax import lax
from jax.experimental import pallas as pl
from jax.experimental.pallas import tpu as pltpu
from jax.experimental.pallas import tpu_sc as plsc

N = 2097152
NW = 32
PER_W = N // NW
C = 32768
T = PER_W // C

MAX_CELL = 256.0 - 2.0 ** -15

ROWS = N // 128
BLK = 1024


def _idx_kernel(x_ref, o_ref):
    def cell(v):
        return jnp.minimum(v * 128.0 + 128.0, MAX_CELL).astype(jnp.int32)

    i = cell(x_ref[:, 0, :])
    j = cell(x_ref[:, 1, :])
    k = cell(x_ref[:, 2, :])
    jpart = (j << 7) + ((j >> 3) << 10)
    kpart = (k & 127) | ((k >> 7) << 10)
    o_ref[:, :] = (i << 16) | jpart | kpart


_compute_idx = pl.pallas_call(
    _idx_kernel,
    grid=(ROWS // BLK,),
    in_specs=[pl.BlockSpec((BLK, 3, 128), lambda g: (g, 0, 0))],
    out_specs=pl.BlockSpec((BLK, 128), lambda g: (g, 0)),
    out_shape=jax.ShapeDtypeStruct((ROWS, 128), jnp.int32),
)

_mesh = plsc.VectorSubcoreMesh(core_axis_name="c", subcore_axis_name="s")


@functools.partial(
    pl.kernel,
    mesh=_mesh,
    out_type=jax.ShapeDtypeStruct((N,), jnp.float32),
    compiler_params=pltpu.CompilerParams(needs_layout_passes=False),
    scratch_types=[
        pltpu.VMEM((C,), jnp.int32),
        pltpu.VMEM((C,), jnp.float32),
    ],
)
def _sc_gather(idx_hbm, grid_hbm, out_hbm, idxv, outv):
    wid = lax.axis_index("s") * 2 + lax.axis_index("c")
    base = wid * PER_W
    for t in range(T):
        off = base + t * C
        pltpu.sync_copy(idx_hbm.at[pl.ds(off, C)], idxv)
        pltpu.sync_copy(grid_hbm.at[idxv], outv)
        pltpu.sync_copy(outv, out_hbm.at[pl.ds(off, C)])


def kernel(x, grid):
    grid_lin = (
        grid.reshape(256, 32, 8, 2, 128)
        .transpose(0, 1, 3, 2, 4)
        .reshape(-1)
    )
    xt = x.reshape(-1, 128, 3).transpose(0, 2, 1)
    idx = _compute_idx(xt).reshape(-1)
    return _sc_gather(idx, grid_lin)


# speedup vs baseline: 17.8844x; 1.0022x over previous
import functools

import jax
import jax.numpy as jnp
from jax import lax
from jax.experimental import pallas as pl
from jax.experimental.pallas import tpu as pltpu
from jax.experimental.pallas import tpu_sc as plsc

N = 2097152
NW = 32
PER_W = N // NW
C = 16384
T = PER_W // C

MAX_CELL = 256.0 - 2.0 ** -15

ROWS = N // 128
BLK = 1024


def _idx_kernel(x_ref, o_ref):
    def cell(v):
        return jnp.minimum(v * 128.0 + 128.0, MAX_CELL).astype(jnp.int32)

    i = cell(x_ref[:, 0, :])
    j = cell(x_ref[:, 1, :])
    k = cell(x_ref[:, 2, :])
    jpart = (j << 7) + ((j >> 3) << 10)
    kpart = (k & 127) | ((k >> 7) << 10)
    o_ref[:, :] = (i << 16) | jpart | kpart


_compute_idx = pl.pallas_call(
    _idx_kernel,
    grid=(ROWS // BLK,),
    in_specs=[pl.BlockSpec((BLK, 3, 128), lambda g: (g, 0, 0))],
    out_specs=pl.BlockSpec((BLK, 128), lambda g: (g, 0)),
    out_shape=jax.ShapeDtypeStruct((ROWS, 128), jnp.int32),
)

_mesh = plsc.VectorSubcoreMesh(core_axis_name="c", subcore_axis_name="s")


@functools.partial(
    pl.kernel,
    mesh=_mesh,
    out_type=jax.ShapeDtypeStruct((N,), jnp.float32),
    compiler_params=pltpu.CompilerParams(needs_layout_passes=False),
    scratch_types=[
        pltpu.VMEM((C,), jnp.int32),
        pltpu.VMEM((C,), jnp.int32),
        pltpu.VMEM((C,), jnp.float32),
        pltpu.VMEM((C,), jnp.float32),
        pltpu.SemaphoreType.DMA,
        pltpu.SemaphoreType.DMA,
        pltpu.SemaphoreType.DMA,
        pltpu.SemaphoreType.DMA,
        pltpu.SemaphoreType.DMA,
        pltpu.SemaphoreType.DMA,
    ],
)
def _sc_gather(idx_hbm, grid_hbm, out_hbm,
               i0, i1, o0, o1, si0, si1, sg0, sg1, so0, so1):
    wid = lax.axis_index("s") * 2 + lax.axis_index("c")
    base = wid * PER_W
    ib = (i0, i1)
    ob = (o0, o1)
    si = (si0, si1)
    sg = (sg0, sg1)
    so = (so0, so1)

    def icopy(t):
        return pltpu.make_async_copy(
            idx_hbm.at[pl.ds(base + t * C, C)], ib[t % 2], si[t % 2])

    def gcopy(t):
        return pltpu.make_async_copy(grid_hbm.at[ib[t % 2]], ob[t % 2], sg[t % 2])

    def ocopy(t):
        return pltpu.make_async_copy(
            ob[t % 2], out_hbm.at[pl.ds(base + t * C, C)], so[t % 2])

    icopy(0).start()
    icopy(1).start()
    for t in range(T):
        icopy(t).wait()
        if t >= 2:
            ocopy(t - 2).wait()
        gcopy(t).start()
        gcopy(t).wait()
        if t + 2 < T:
            icopy(t + 2).start()
        ocopy(t).start()
    ocopy(T - 2).wait()
    ocopy(T - 1).wait()


def kernel(x, grid):
    grid_lin = (
        grid.reshape(256, 32, 8, 2, 128)
        .transpose(0, 1, 3, 2, 4)
        .reshape(-1)
    )
    xt = x.reshape(-1, 128, 3).transpose(0, 2, 1)
    idx = _compute_idx(xt).reshape(-1)
    return _sc_gather(idx, grid_lin)
